# baseline (speedup 1.0000x reference)
import jax
import jax.numpy as jnp
from jax.experimental import pallas as pl


def _mp(edge_index, x, norm_elev, norm_length, norm_geom_1, norm_in_offset, norm_out_offset, W, b):
    num_nodes = x.shape[0]
    src = edge_index[0]
    dst = edge_index[1]
    h_src = x[src] + norm_elev[src] + norm_out_offset
    h_dst = x[dst] + norm_elev[dst] + norm_in_offset
    m_in = jnp.concatenate([h_src, h_dst, norm_length, norm_geom_1], axis=1)
    m = jax.nn.relu(m_in @ W + b)
    return jax.ops.segment_sum(m, dst, num_segments=num_nodes)


def _identity_kernel(x_ref, o_ref):
    o_ref[...] = x_ref[...]


def kernel(x, edge_index, norm_elev, norm_length, norm_geom_1, norm_in_offset, norm_out_offset, W1, b1, W2, b2):
    steps_ahead = x.shape[1] - 1
    cur_x = x
    preds = []
    for step in range(steps_ahead):
        one_step_x = cur_x[:, :2]
        out_mp = _mp(edge_index, one_step_x, norm_elev, norm_length, norm_geom_1, norm_in_offset, norm_out_offset, W1, b1)
        y = _mp(edge_index, out_mp, norm_elev, norm_length, norm_geom_1, norm_in_offset, norm_out_offset, W2, b2)
        preds.append(y.reshape(-1))
        new_runoff = cur_x[:, 2:]
        cur_x = jnp.concatenate([y, new_runoff], axis=1)
    out = jnp.stack(preds, axis=1)
    return pl.pallas_call(
        _identity_kernel,
        out_shape=jax.ShapeDtypeStruct(out.shape, out.dtype),
        grid=(100,),
        in_specs=[pl.BlockSpec((out.shape[0] // 100, out.shape[1]), lambda i: (i, 0))],
        out_specs=pl.BlockSpec((out.shape[0] // 100, out.shape[1]), lambda i: (i, 0)),
    )(out)


# speedup vs baseline: 16.4660x; 16.4660x over previous
import jax
import jax.numpy as jnp
from jax import lax
from jax.experimental import pallas as pl
from jax.experimental.pallas import tpu as pltpu
from jax.experimental.pallas import tpu_sc as plsc

NP = 102400
N_TILES = 16
NODES_PER_TILE = NP // N_TILES
EROWS = 12800
E_PAD = EROWS * 128
ROWS_PER_TILE = EROWS // N_TILES
CH_ROWS = 32
N_CHUNKS = ROWS_PER_TILE // CH_ROWS
STEPS = 8

F32 = jnp.float32
I32 = jnp.int32


def _bf16r(v):
    return v.astype(jnp.bfloat16).astype(F32)


def _body(xtf, src, dst, ln, gm, io, oo, elev, wt,
          outf, c1a, c1b, c1c, c2, ese, ede,
          s_elev, s_ns0, s_ns1, s_mp0, s_mp1, s_mp2, s_y,
          v_sidx, v_didx,
          v_a, v_b, v_c, v_d, v_e, v_f, v_g, v_h, v_i, v_j, v_k, v_l,
          v_nbuf, v_zbuf, v_wtab, sem):
    wid = lax.axis_index("s")
    nbase = pl.multiple_of(wid * NODES_PER_TILE, NODES_PER_TILE)
    erow0 = wid * ROWS_PER_TILE
    nsl = pl.ds(nbase, NODES_PER_TILE)

    def wrow(r):
        return v_wtab[pl.ds(16 * r, 16)]

    pltpu.sync_copy(wt, v_wtab)

    def zinit(i, _):
        v_zbuf[pl.ds(pl.multiple_of(i * 16, 16), 16)] = jnp.zeros((16,), F32)
        return 0
    lax.fori_loop(0, NODES_PER_TILE // 16, zinit, 0)

    pltpu.sync_copy(elev.at[nsl], s_elev.at[nsl])
    pltpu.sync_copy(xtf.at[nsl], s_ns0.at[nsl])
    plsc.subcore_barrier()

    def phase0(c, _):
        row0 = erow0 + c * CH_ROWS
        rsl = pl.ds(row0, CH_ROWS)
        pltpu.sync_copy(src.at[rsl], v_sidx)
        pltpu.sync_copy(dst.at[rsl], v_didx)
        pltpu.sync_copy(ln.at[rsl], v_a)
        pltpu.sync_copy(gm.at[rsl], v_b)
        pltpu.sync_copy(io.at[rsl], v_c)
        pltpu.sync_copy(oo.at[rsl], v_d)

        def grow(j, _):
            d1 = pltpu.async_copy(s_elev.at[v_sidx.at[j]], v_e.at[j], sem)
            d2 = pltpu.async_copy(s_elev.at[v_didx.at[j]], v_f.at[j], sem)
            d1.wait()
            d2.wait()
            return 0
        lax.fori_loop(0, CH_ROWS, grow, 0)

        def comp(i, _):
            def comp16(k, _):
                csl = pl.ds(pl.multiple_of(k * 16, 16), 16)
                lnr = _bf16r(v_a[i, csl])
                gmr = _bf16r(v_b[i, csl])
                v_g[i, csl] = lnr * wrow(18) + gmr * wrow(19) + wrow(20)
                v_h[i, csl] = lnr * wrow(21) + gmr * wrow(22) + wrow(23)
                v_i[i, csl] = lnr * wrow(24) + gmr * wrow(25) + wrow(26)
                v_j[i, csl] = lnr * wrow(27) + gmr * wrow(28) + wrow(29)
                v_k[i, csl] = v_e[i, csl] + v_d[i, csl]
                v_l[i, csl] = v_f[i, csl] + v_c[i, csl]
                return 0
            lax.fori_loop(0, 8, comp16, 0)
            return 0
        lax.fori_loop(0, CH_ROWS, comp, 0)

        pltpu.sync_copy(v_g, c1a.at[rsl])
        pltpu.sync_copy(v_h, c1b.at[rsl])
        pltpu.sync_copy(v_i, c1c.at[rsl])
        pltpu.sync_copy(v_j, c2.at[rsl])
        pltpu.sync_copy(v_k, ese.at[rsl])
        pltpu.sync_copy(v_l, ede.at[rsl])
        return 0
    lax.fori_loop(0, N_CHUNKS, phase0, 0)

    def step(s, _):
        xoff = pl.multiple_of((s + 1) * NP, NP) + nbase
        pltpu.sync_copy(xtf.at[pl.ds(xoff, NODES_PER_TILE)], s_ns1.at[nsl])
        pltpu.sync_copy(v_zbuf, s_mp0.at[nsl])
        pltpu.sync_copy(v_zbuf, s_mp1.at[nsl])
        pltpu.sync_copy(v_zbuf, s_mp2.at[nsl])
        pltpu.sync_copy(v_zbuf, s_y.at[nsl])
        plsc.subcore_barrier()

        def pass1(c, _):
            row0 = erow0 + c * CH_ROWS
            rsl = pl.ds(row0, CH_ROWS)
            pltpu.sync_copy(src.at[rsl], v_sidx)
            pltpu.sync_copy(dst.at[rsl], v_didx)
            pltpu.sync_copy(c1a.at[rsl], v_e)
            pltpu.sync_copy(c1b.at[rsl], v_f)
            pltpu.sync_copy(c1c.at[rsl], v_g)
            pltpu.sync_copy(ese.at[rsl], v_k)
            pltpu.sync_copy(ede.at[rsl], v_l)

            def grow(j, _):
                sj = v_sidx.at[j]
                dj = v_didx.at[j]
                d1 = pltpu.async_copy(s_ns0.at[sj], v_a.at[j], sem)
                d2 = pltpu.async_copy(s_ns1.at[sj], v_b.at[j], sem)
                d3 = pltpu.async_copy(s_ns0.at[dj], v_c.at[j], sem)
                d4 = pltpu.async_copy(s_ns1.at[dj], v_d.at[j], sem)
                d1.wait()
                d2.wait()
                d3.wait()
                d4.wait()
                return 0
            lax.fori_loop(0, CH_ROWS, grow, 0)

            def comp(i, _):
                def comp16(k, _):
                    csl = pl.ds(pl.multiple_of(k * 16, 16), 16)
                    es = v_k[i, csl]
                    ed = v_l[i, csl]
                    hs0 = _bf16r(v_a[i, csl] + es)
                    hs1 = _bf16r(v_b[i, csl] + es)
                    hd0 = _bf16r(v_c[i, csl] + ed)
                    hd1 = _bf16r(v_d[i, csl] + ed)
                    z = jnp.zeros((16,), F32)
                    v_h[i, csl] = jnp.maximum(
                        hs0 * wrow(0) + hs1 * wrow(1) + hd0 * wrow(2)
                        + hd1 * wrow(3) + v_e[i, csl], z)
                    v_i[i, csl] = jnp.maximum(
                        hs0 * wrow(4) + hs1 * wrow(5) + hd0 * wrow(6)
                        + hd1 * wrow(7) + v_f[i, csl], z)
                    v_j[i, csl] = jnp.maximum(
                        hs0 * wrow(8) + hs1 * wrow(9) + hd0 * wrow(10)
                        + hd1 * wrow(11) + v_g[i, csl], z)
                    return 0
                lax.fori_loop(0, 8, comp16, 0)
                return 0
            lax.fori_loop(0, CH_ROWS, comp, 0)

            def srow(j, _):
                dj = v_didx.at[j]
                pltpu.async_copy(v_h.at[j], s_mp0.at[dj], sem, add=True).wait()
                pltpu.async_copy(v_i.at[j], s_mp1.at[dj], sem, add=True).wait()
                pltpu.async_copy(v_j.at[j], s_mp2.at[dj], sem, add=True).wait()
                return 0
            lax.fori_loop(0, CH_ROWS, srow, 0)
            return 0
        lax.fori_loop(0, N_CHUNKS, pass1, 0)
        plsc.subcore_barrier()

        def pass2(c, _):
            row0 = erow0 + c * CH_ROWS
            rsl = pl.ds(row0, CH_ROWS)
            pltpu.sync_copy(src.at[rsl], v_sidx)
            pltpu.sync_copy(dst.at[rsl], v_didx)
            pltpu.sync_copy(c2.at[rsl], v_g)
            pltpu.sync_copy(ese.at[rsl], v_k)
            pltpu.sync_copy(ede.at[rsl], v_l)

            def grow(j, _):
                sj = v_sidx.at[j]
                dj = v_didx.at[j]
                d1 = pltpu.async_copy(s_mp0.at[sj], v_a.at[j], sem)
                d2 = pltpu.async_copy(s_mp1.at[sj], v_b.at[j], sem)
                d3 = pltpu.async_copy(s_mp2.at[sj], v_c.at[j], sem)
                d4 = pltpu.async_copy(s_mp0.at[dj], v_d.at[j], sem)
                d5 = pltpu.async_copy(s_mp1.at[dj], v_e.at[j], sem)
                d6 = pltpu.async_copy(s_mp2.at[dj], v_f.at[j], sem)
                d1.wait()
                d2.wait()
                d3.wait()
                d4.wait()
                d5.wait()
                d6.wait()
                return 0
            lax.fori_loop(0, CH_ROWS, grow, 0)

            def comp(i, _):
                def comp16(k, _):
                    csl = pl.ds(pl.multiple_of(k * 16, 16), 16)
                    es = v_k[i, csl]
                    ed = v_l[i, csl]
                    acc = (_bf16r(v_a[i, csl] + es) * wrow(12)
                           + _bf16r(v_b[i, csl] + es) * wrow(13)
                           + _bf16r(v_c[i, csl] + es) * wrow(14)
                           + _bf16r(v_d[i, csl] + ed) * wrow(15)
                           + _bf16r(v_e[i, csl] + ed) * wrow(16)
                           + _bf16r(v_f[i, csl] + ed) * wrow(17)
                           + v_g[i, csl])
                    v_h[i, csl] = jnp.maximum(acc, jnp.zeros((16,), F32))
                    return 0
                lax.fori_loop(0, 8, comp16, 0)
                return 0
            lax.fori_loop(0, CH_ROWS, comp, 0)

            def srow(j, _):
                pltpu.async_copy(v_h.at[j], s_y.at[v_didx.at[j]], sem,
                                 add=True).wait()
                return 0
            lax.fori_loop(0, CH_ROWS, srow, 0)
            return 0
        lax.fori_loop(0, N_CHUNKS, pass2, 0)
        plsc.subcore_barrier()

        pltpu.sync_copy(s_y.at[nsl], v_nbuf)
        oof = pl.multiple_of(s * NP, NP) + nbase
        pltpu.sync_copy(v_nbuf, outf.at[pl.ds(oof, NODES_PER_TILE)])
        pltpu.sync_copy(v_nbuf, s_ns0.at[nsl])
        return 0
    lax.fori_loop(0, STEPS, step, 0)


@jax.jit
def kernel(x, edge_index, norm_elev, norm_length, norm_geom_1,
           norm_in_offset, norm_out_offset, W1, b1, W2, b2):
    N = x.shape[0]
    E = edge_index.shape[1]

    xtf = jnp.pad(x.T, ((0, 0), (0, NP - N))).reshape(-1)
    elev = jnp.pad(norm_elev[:, 0], (0, NP - N))
    src = jnp.pad(edge_index[0], (0, E_PAD - E)).reshape(EROWS, 128)
    dst = jnp.pad(edge_index[1], (0, E_PAD - E),
                  constant_values=N).reshape(EROWS, 128)
    ln = jnp.pad(norm_length[:, 0], (0, E_PAD - E)).reshape(EROWS, 128)
    gm = jnp.pad(norm_geom_1[:, 0], (0, E_PAD - E)).reshape(EROWS, 128)
    io = jnp.pad(norm_in_offset[:, 0], (0, E_PAD - E)).reshape(EROWS, 128)
    oo = jnp.pad(norm_out_offset[:, 0], (0, E_PAD - E)).reshape(EROWS, 128)

    W1r = W1.astype(jnp.bfloat16).astype(F32)
    W2r = W2.astype(jnp.bfloat16).astype(F32)
    coef = []
    for j in range(3):
        coef += [W1r[0, j], W1r[1, j], W1r[2, j], W1r[3, j]]
    coef += [W2r[k, 0] for k in range(6)]
    for j in range(3):
        coef += [W1r[4, j], W1r[5, j], b1[j]]
    coef += [W2r[6, 0], W2r[7, 0], b2[0]]
    wt = jnp.tile(jnp.stack(coef)[:, None], (1, 16)).reshape(-1)

    mesh = plsc.VectorSubcoreMesh(core_axis_name="c", subcore_axis_name="s",
                                  num_cores=1)
    out_type = (
        jax.ShapeDtypeStruct((STEPS * NP,), F32),
        jax.ShapeDtypeStruct((EROWS, 128), F32),
        jax.ShapeDtypeStruct((EROWS, 128), F32),
        jax.ShapeDtypeStruct((EROWS, 128), F32),
        jax.ShapeDtypeStruct((EROWS, 128), F32),
        jax.ShapeDtypeStruct((EROWS, 128), F32),
        jax.ShapeDtypeStruct((EROWS, 128), F32),
    )
    scratch = [
        pltpu.VMEM_SHARED((NP,), F32),
        pltpu.VMEM_SHARED((NP,), F32),
        pltpu.VMEM_SHARED((NP,), F32),
        pltpu.VMEM_SHARED((NP,), F32),
        pltpu.VMEM_SHARED((NP,), F32),
        pltpu.VMEM_SHARED((NP,), F32),
        pltpu.VMEM_SHARED((NP,), F32),
        pltpu.VMEM((CH_ROWS, 128), I32),
        pltpu.VMEM((CH_ROWS, 128), I32),
    ] + [pltpu.VMEM((CH_ROWS, 128), F32) for _ in range(12)] + [
        pltpu.VMEM((NODES_PER_TILE,), F32),
        pltpu.VMEM((NODES_PER_TILE,), F32),
        pltpu.VMEM((480,), F32),
        pltpu.SemaphoreType.DMA,
    ]
    outs = pl.kernel(
        _body, out_type=out_type, mesh=mesh, scratch_types=scratch,
    )(xtf, src, dst, ln, gm, io, oo, elev, wt)
    return outs[0].reshape(STEPS, NP)[:, :N].T
